# baseline (speedup 1.0000x reference)
import functools

import jax
import jax.numpy as jnp
from jax import lax
from jax.experimental import pallas as pl
from jax.experimental.pallas import tpu as pltpu
from jax.experimental.pallas import tpu_sc as plsc

_BM = 256
_NB = 16
_NP = _BM * _NB
_NW = 32


def _argmax_rows(logits, Eo):
    m = jnp.max(logits, axis=-1, keepdims=True)
    iot = lax.broadcasted_iota(jnp.int32, logits.shape, 1)
    return jnp.min(jnp.where(logits == m, iot, Eo), axis=-1, keepdims=True)


def _router_argmax(xin, Wr, br, nrows):
    D = xin.shape[1]
    Eo = Wr.shape[1]
    BM = 256

    def kern(x_ref, w_ref, b_ref, o_ref):
        logits = (
            jnp.dot(x_ref[...], w_ref[...], preferred_element_type=jnp.float32)
            + b_ref[...]
        )
        o_ref[...] = _argmax_rows(logits, Eo)

    return pl.pallas_call(
        kern,
        grid=(nrows // BM,),
        in_specs=[
            pl.BlockSpec((BM, D), lambda i: (i, 0)),
            pl.BlockSpec((D, Eo), lambda i: (0, 0)),
            pl.BlockSpec((1, Eo), lambda i: (0, 0)),
        ],
        out_specs=pl.BlockSpec((BM, 1), lambda i: (i, 0)),
        out_shape=jax.ShapeDtypeStruct((nrows, 1), jnp.int32),
    )(xin, Wr, br.reshape(1, Eo))


def _grouped_matmul0(xg, We, be, Wr, br, emap, bvalid, bclamp):
    NPr, K = xg.shape
    E, _, P = We.shape
    Eo = Wr.shape[1]
    BN = 1024
    NJ = P // BN
    NBg = NPr // _BM

    def kern(emap_ref, bvalid_ref, bclamp_ref, x_ref, w_ref, b_ref, wr_ref,
             br_ref, o_ref, a_ref, acc_ref):
        j = pl.program_id(0)
        b = pl.program_id(1)

        @pl.when(bvalid_ref[b] == 1)
        def _():
            acc = jnp.maximum(
                jnp.dot(x_ref[...], w_ref[0], preferred_element_type=jnp.float32)
                + b_ref[0],
                0.0,
            )
            o_ref[...] = acc
            lg = jnp.dot(acc, wr_ref[...], preferred_element_type=jnp.float32)
            sl = pl.ds(b * _BM, _BM)

            @pl.when(j == 0)
            def _():
                acc_ref[sl, :] = lg + br_ref[...]

            @pl.when(j > 0)
            def _():
                acc_ref[sl, :] += lg

            @pl.when(j == NJ - 1)
            def _():
                a_ref[...] = _argmax_rows(acc_ref[sl, :], Eo)

    grid_spec = pltpu.PrefetchScalarGridSpec(
        num_scalar_prefetch=3,
        grid=(NJ, NBg),
        in_specs=[
            pl.BlockSpec((_BM, K), lambda j, b, em, bv, bc: (bc[b], 0)),
            pl.BlockSpec((1, K, BN), lambda j, b, em, bv, bc: (em[b], 0, j)),
            pl.BlockSpec((1, 1, BN), lambda j, b, em, bv, bc: (em[b], 0, j)),
            pl.BlockSpec((BN, Eo), lambda j, b, em, bv, bc: (j, 0)),
            pl.BlockSpec((1, Eo), lambda j, b, em, bv, bc: (0, 0)),
        ],
        out_specs=[
            pl.BlockSpec((_BM, BN), lambda j, b, em, bv, bc: (bc[b], j)),
            pl.BlockSpec((_BM, 1), lambda j, b, em, bv, bc: (bc[b], 0)),
        ],
        scratch_shapes=[pltpu.VMEM((NPr, Eo), jnp.float32)],
    )
    return pl.pallas_call(
        kern,
        grid_spec=grid_spec,
        out_shape=[
            jax.ShapeDtypeStruct((2 * NPr, P), jnp.float32),
            jax.ShapeDtypeStruct((NPr, 1), jnp.int32),
        ],
    )(emap, bvalid, bclamp, xg, We, be.reshape(E, 1, P), Wr,
      br.reshape(1, Eo))


def _grouped_matmul1(xg, We, be, emap, bvalid, bclamp, alias_in):
    NPr, K = xg.shape
    E, _, P = We.shape
    BN = 1024
    NBg = NPr // _BM

    def kern(emap_ref, bvalid_ref, bclamp_ref, x_ref, w_ref, b_ref, big_ref,
             o_ref):
        @pl.when(bvalid_ref[pl.program_id(1)] == 1)
        def _():
            o_ref[...] = (
                jnp.dot(x_ref[...], w_ref[0], preferred_element_type=jnp.float32)
                + b_ref[0]
            )

    grid_spec = pltpu.PrefetchScalarGridSpec(
        num_scalar_prefetch=3,
        grid=(P // BN, NBg),
        in_specs=[
            pl.BlockSpec((_BM, K), lambda j, b, em, bv, bc: (bc[b], 0)),
            pl.BlockSpec((1, K, BN), lambda j, b, em, bv, bc: (em[b], 0, j)),
            pl.BlockSpec((1, 1, BN), lambda j, b, em, bv, bc: (em[b], 0, j)),
            pl.BlockSpec(memory_space=pl.ANY),
        ],
        out_specs=pl.BlockSpec((_BM, BN), lambda j, b, em, bv, bc: (bc[b] + NBg, j)),
    )
    return pl.pallas_call(
        kern,
        grid_spec=grid_spec,
        out_shape=jax.ShapeDtypeStruct((2 * NPr, P), jnp.float32),
        input_output_aliases={6: 0},
    )(emap, bvalid, bclamp, xg, We, be.reshape(E, 1, P), alias_in)


def _sc_gather(table, idx, meta, out_rows, CH):
    D = table.shape[1]
    mesh = plsc.VectorSubcoreMesh(core_axis_name="c", subcore_axis_name="s")

    @functools.partial(
        pl.kernel,
        mesh=mesh,
        out_type=jax.ShapeDtypeStruct((out_rows, D), jnp.float32),
        scratch_types=[
            pltpu.VMEM((16,), jnp.float32),
            pltpu.VMEM((CH,), jnp.int32),
            pltpu.VMEM((CH,), jnp.int32),
            pltpu.VMEM((CH, D), jnp.float32),
            pltpu.VMEM((CH, D), jnp.float32),
            pltpu.SemaphoreType.DMA,
            pltpu.SemaphoreType.DMA,
        ],
    )
    def gather_k(table_hbm, idx_hbm, meta_hbm, out_hbm,
                 meta_v, idxA, idxB, rowsA, rowsB, semA, semB):
        wid = lax.axis_index("s") * 2 + lax.axis_index("c")
        pltpu.sync_copy(meta_hbm, meta_v)
        total_ch = meta_v[...][0].astype(jnp.int32)
        nloc = (total_ch - wid + _NW - 1) // _NW
        npairs = (nloc + 1) // 2

        def pair(j, carry):
            k0 = 2 * j
            k1 = 2 * j + 1
            off0 = (wid + k0 * _NW) * CH
            off1 = (wid + k1 * _NW) * CH
            pltpu.sync_copy(idx_hbm.at[pl.ds(off0, CH)], idxA)
            cA = pltpu.async_copy(table_hbm.at[idxA], rowsA, semA)

            @pl.when(k1 < nloc)
            def _():
                pltpu.sync_copy(idx_hbm.at[pl.ds(off1, CH)], idxB)
                pltpu.async_copy(table_hbm.at[idxB], rowsB, semB)

            cA.wait()
            pltpu.sync_copy(rowsA, out_hbm.at[pl.ds(off0, CH)])

            @pl.when(k1 < nloc)
            def _():
                pltpu.make_async_copy(table_hbm.at[idxB], rowsB, semB).wait()
                pltpu.sync_copy(rowsB, out_hbm.at[pl.ds(off1, CH)])

            return carry

        lax.fori_loop(0, npairs, pair, 0)

    return gather_k(table, idx, meta)


def _dispatch(act, valid, rows, E):
    T = act.shape[0]
    oh = (act[:, None] == jnp.arange(E, dtype=act.dtype)[None, :]) & valid[:, None]
    ohi = oh.astype(jnp.int32)
    counts = jnp.sum(ohi, axis=0)
    nb = (counts + _BM - 1) // _BM
    bstart = jnp.cumsum(nb) - nb
    rank = jnp.take_along_axis(jnp.cumsum(ohi, axis=0) - 1, act[:, None], axis=1)[:, 0]
    p = bstart[act] * _BM + rank
    p_sc = jnp.where(valid, p, _NP)
    g = (jnp.arange(_NP, dtype=jnp.int32) % T).at[p_sc].set(
        rows.astype(jnp.int32), mode="drop")
    nvalid = jnp.sum(nb)
    bid = jnp.arange(_NB)
    bexp = jnp.clip(jnp.searchsorted(bstart, bid, side="right") - 1, 0, E - 1)
    last = bexp[jnp.clip(nvalid - 1, 0, _NB - 1)]
    bexp = jnp.where(bid < nvalid, bexp, last).astype(jnp.int32)
    bvalid = (bid < nvalid).astype(jnp.int32)
    bclamp = jnp.minimum(bid, jnp.maximum(nvalid - 1, 0)).astype(jnp.int32)
    return g, p.astype(jnp.int32), bexp, bvalid, bclamp, nvalid


def _meta16(total_chunks):
    return jnp.zeros((16,), jnp.float32).at[0].set(
        jnp.asarray(total_chunks).astype(jnp.float32))


def _i2h_matmul(x, W, b):
    T, D = x.shape
    P = W.shape[1]
    BM, BN = 256, 1024

    def kern(x_ref, w_ref, b_ref, o_ref):
        o_ref[...] = (
            jnp.dot(x_ref[...], w_ref[...], preferred_element_type=jnp.float32)
            + b_ref[...]
        )

    return pl.pallas_call(
        kern,
        grid=(P // BN, T // BM),
        in_specs=[
            pl.BlockSpec((BM, D), lambda j, i: (i, 0)),
            pl.BlockSpec((D, BN), lambda j, i: (0, j)),
            pl.BlockSpec((1, BN), lambda j, i: (0, j)),
        ],
        out_specs=pl.BlockSpec((BM, BN), lambda j, i: (i, j)),
        out_shape=jax.ShapeDtypeStruct((T, P), jnp.float32),
    )(x, W, b.reshape(1, P))


def _lstm(i2h, h2h, c):
    T, P = i2h.shape
    H = c.shape[1]
    BM = 256

    def kern(i2h_ref, h2h_ref, c_ref, o_ref):
        gates = i2h_ref[...] + h2h_ref[...]
        i_g = gates[:, 0:H]
        f_g = gates[:, H:2 * H]
        g_g = gates[:, 2 * H:3 * H]
        o_g = gates[:, 3 * H:4 * H]
        c_new = jax.nn.sigmoid(f_g) * c_ref[...] + jax.nn.sigmoid(i_g) * jnp.tanh(g_g)
        h_new = jax.nn.sigmoid(o_g) * jnp.tanh(c_new)
        o_ref[:, 0:H] = h_new
        o_ref[:, H:2 * H] = c_new

    return pl.pallas_call(
        kern,
        grid=(T // BM,),
        in_specs=[
            pl.BlockSpec((BM, P), lambda i: (i, 0)),
            pl.BlockSpec((BM, P), lambda i: (i, 0)),
            pl.BlockSpec((BM, H), lambda i: (i, 0)),
        ],
        out_specs=pl.BlockSpec((BM, 2 * H), lambda i: (i, 0)),
        out_shape=jax.ShapeDtypeStruct((T, 2 * H), jnp.float32),
    )(i2h, h2h, c)


def kernel(x, h, c, W_i2h, b_i2h, Wr0, br0, We0, be0, Wr1, br1, We1, be1):
    T, H = h.shape
    E = We0.shape[0]

    i2h = _i2h_matmul(x, W_i2h, b_i2h)

    a0 = _router_argmax(h, Wr0, br0, T)[:, 0]
    tok = jnp.arange(T, dtype=jnp.int32)
    g0, p0, emap0, bval0, bcl0, nval0 = _dispatch(
        a0, jnp.ones((T,), jnp.bool_), tok, E)

    hg = _sc_gather(h, g0, _meta16(nval0 * (_BM // 64)), _NP, CH=64)
    big, a1g = _grouped_matmul0(hg, We0, be0, Wr1, br1, emap0, bval0, bcl0)

    a1 = a1g[p0, 0]
    maskv = a0 != 0

    g1, p1, emap1, bval1, bcl1, nval1 = _dispatch(a1, maskv, p0, E)
    ig = _sc_gather(big, g1, _meta16(nval1 * (_BM // 16)), _NP, CH=16)
    big = _grouped_matmul1(ig, We1, be1, emap1, bval1, bcl1, alias_in=big)

    src = jnp.where(maskv, _NP + p1, p0)
    h2h = _sc_gather(big, src, _meta16(jnp.asarray(T // 16)), T, CH=16)
    return _lstm(i2h, h2h, c)


# speedup vs baseline: 1.1131x; 1.1131x over previous
import functools

import jax
import jax.numpy as jnp
from jax import lax
from jax.experimental import pallas as pl
from jax.experimental.pallas import tpu as pltpu
from jax.experimental.pallas import tpu_sc as plsc

_BM = 256
_NB = 16
_NP = _BM * _NB
_NW = 32


def _argmax_rows(logits, Eo):
    m = jnp.max(logits, axis=-1, keepdims=True)
    iot = lax.broadcasted_iota(jnp.int32, logits.shape, 1)
    return jnp.min(jnp.where(logits == m, iot, Eo), axis=-1, keepdims=True)


def _router_argmax(xin, Wr, br, nrows):
    D = xin.shape[1]
    Eo = Wr.shape[1]
    BM = 256

    def kern(x_ref, w_ref, b_ref, o_ref):
        logits = (
            jnp.dot(x_ref[...], w_ref[...], preferred_element_type=jnp.float32)
            + b_ref[...]
        )
        o_ref[...] = _argmax_rows(logits, Eo)

    return pl.pallas_call(
        kern,
        grid=(nrows // BM,),
        in_specs=[
            pl.BlockSpec((BM, D), lambda i: (i, 0)),
            pl.BlockSpec((D, Eo), lambda i: (0, 0)),
            pl.BlockSpec((1, Eo), lambda i: (0, 0)),
        ],
        out_specs=pl.BlockSpec((BM, 1), lambda i: (i, 0)),
        out_shape=jax.ShapeDtypeStruct((nrows, 1), jnp.int32),
    )(xin, Wr, br.reshape(1, Eo))


def _grouped_matmul0(xg, We, be, Wr, br, emap, bvalid, bclamp):
    NPr, K = xg.shape
    E, _, P = We.shape
    Eo = Wr.shape[1]
    BN = 1536
    NJ = P // BN
    NBg = NPr // _BM

    def kern(emap_ref, bvalid_ref, bclamp_ref, x_ref, w_ref, b_ref, wr_ref,
             br_ref, o_ref, a_ref, acc_ref):
        j = pl.program_id(0)
        b = pl.program_id(1)

        @pl.when(bvalid_ref[b] == 1)
        def _():
            acc = jnp.maximum(
                jnp.dot(x_ref[...], w_ref[0], preferred_element_type=jnp.float32)
                + b_ref[0],
                0.0,
            )
            o_ref[...] = acc
            lg = jnp.dot(acc, wr_ref[...], preferred_element_type=jnp.float32)
            sl = pl.ds(b * _BM, _BM)

            @pl.when(j == 0)
            def _():
                acc_ref[sl, :] = lg + br_ref[...]

            @pl.when(j > 0)
            def _():
                acc_ref[sl, :] += lg

            @pl.when(j == NJ - 1)
            def _():
                a_ref[...] = _argmax_rows(acc_ref[sl, :], Eo)

    grid_spec = pltpu.PrefetchScalarGridSpec(
        num_scalar_prefetch=3,
        grid=(NJ, NBg),
        in_specs=[
            pl.BlockSpec((_BM, K), lambda j, b, em, bv, bc: (bc[b], 0)),
            pl.BlockSpec((1, K, BN), lambda j, b, em, bv, bc: (em[b], 0, j)),
            pl.BlockSpec((1, 1, BN), lambda j, b, em, bv, bc: (em[b], 0, j)),
            pl.BlockSpec((BN, Eo), lambda j, b, em, bv, bc: (j, 0)),
            pl.BlockSpec((1, Eo), lambda j, b, em, bv, bc: (0, 0)),
        ],
        out_specs=[
            pl.BlockSpec((_BM, BN), lambda j, b, em, bv, bc: (bc[b], j)),
            pl.BlockSpec((_BM, 1), lambda j, b, em, bv, bc: (bc[b], 0)),
        ],
        scratch_shapes=[pltpu.VMEM((NPr, Eo), jnp.float32)],
    )
    return pl.pallas_call(
        kern,
        grid_spec=grid_spec,
        out_shape=[
            jax.ShapeDtypeStruct((2 * NPr, P), jnp.float32),
            jax.ShapeDtypeStruct((NPr, 1), jnp.int32),
        ],
    )(emap, bvalid, bclamp, xg, We, be.reshape(E, 1, P), Wr,
      br.reshape(1, Eo))


def _grouped_matmul1(xg, We, be, emap, bvalid, bclamp, alias_in):
    NPr, K = xg.shape
    E, _, P = We.shape
    BN = 1536
    NBg = NPr // _BM

    def kern(emap_ref, bvalid_ref, bclamp_ref, x_ref, w_ref, b_ref, big_ref,
             o_ref):
        @pl.when(bvalid_ref[pl.program_id(1)] == 1)
        def _():
            o_ref[...] = (
                jnp.dot(x_ref[...], w_ref[0], preferred_element_type=jnp.float32)
                + b_ref[0]
            )

    grid_spec = pltpu.PrefetchScalarGridSpec(
        num_scalar_prefetch=3,
        grid=(P // BN, NBg),
        in_specs=[
            pl.BlockSpec((_BM, K), lambda j, b, em, bv, bc: (bc[b], 0)),
            pl.BlockSpec((1, K, BN), lambda j, b, em, bv, bc: (em[b], 0, j)),
            pl.BlockSpec((1, 1, BN), lambda j, b, em, bv, bc: (em[b], 0, j)),
            pl.BlockSpec(memory_space=pl.ANY),
        ],
        out_specs=pl.BlockSpec((_BM, BN), lambda j, b, em, bv, bc: (bc[b] + NBg, j)),
    )
    return pl.pallas_call(
        kern,
        grid_spec=grid_spec,
        out_shape=jax.ShapeDtypeStruct((2 * NPr, P), jnp.float32),
        input_output_aliases={6: 0},
    )(emap, bvalid, bclamp, xg, We, be.reshape(E, 1, P), alias_in)


def _sc_gather(table, idx, meta, out_rows, CH):
    D = table.shape[1]
    mesh = plsc.VectorSubcoreMesh(core_axis_name="c", subcore_axis_name="s")

    @functools.partial(
        pl.kernel,
        mesh=mesh,
        out_type=jax.ShapeDtypeStruct((out_rows, D), jnp.float32),
        scratch_types=[
            pltpu.VMEM((16,), jnp.float32),
            pltpu.VMEM((CH,), jnp.int32),
            pltpu.VMEM((CH,), jnp.int32),
            pltpu.VMEM((CH, D), jnp.float32),
            pltpu.VMEM((CH, D), jnp.float32),
            pltpu.SemaphoreType.DMA,
            pltpu.SemaphoreType.DMA,
        ],
    )
    def gather_k(table_hbm, idx_hbm, meta_hbm, out_hbm,
                 meta_v, idxA, idxB, rowsA, rowsB, semA, semB):
        wid = lax.axis_index("s") * 2 + lax.axis_index("c")
        pltpu.sync_copy(meta_hbm, meta_v)
        total_ch = meta_v[...][0].astype(jnp.int32)
        nloc = (total_ch - wid + _NW - 1) // _NW
        npairs = (nloc + 1) // 2

        def pair(j, carry):
            k0 = 2 * j
            k1 = 2 * j + 1
            off0 = (wid + k0 * _NW) * CH
            off1 = (wid + k1 * _NW) * CH
            pltpu.sync_copy(idx_hbm.at[pl.ds(off0, CH)], idxA)
            cA = pltpu.async_copy(table_hbm.at[idxA], rowsA, semA)

            @pl.when(k1 < nloc)
            def _():
                pltpu.sync_copy(idx_hbm.at[pl.ds(off1, CH)], idxB)
                pltpu.async_copy(table_hbm.at[idxB], rowsB, semB)

            cA.wait()
            pltpu.sync_copy(rowsA, out_hbm.at[pl.ds(off0, CH)])

            @pl.when(k1 < nloc)
            def _():
                pltpu.make_async_copy(table_hbm.at[idxB], rowsB, semB).wait()
                pltpu.sync_copy(rowsB, out_hbm.at[pl.ds(off1, CH)])

            return carry

        lax.fori_loop(0, npairs, pair, 0)

    return gather_k(table, idx, meta)


def _dispatch(act, valid, rows, E):
    T = act.shape[0]
    oh = (act[:, None] == jnp.arange(E, dtype=act.dtype)[None, :]) & valid[:, None]
    ohi = oh.astype(jnp.int32)
    counts = jnp.sum(ohi, axis=0)
    nb = (counts + _BM - 1) // _BM
    bstart = jnp.cumsum(nb) - nb
    rank = jnp.take_along_axis(jnp.cumsum(ohi, axis=0) - 1, act[:, None], axis=1)[:, 0]
    p = bstart[act] * _BM + rank
    p_sc = jnp.where(valid, p, _NP)
    g = (jnp.arange(_NP, dtype=jnp.int32) % T).at[p_sc].set(
        rows.astype(jnp.int32), mode="drop")
    nvalid = jnp.sum(nb)
    bid = jnp.arange(_NB)
    bexp = jnp.clip(jnp.searchsorted(bstart, bid, side="right") - 1, 0, E - 1)
    last = bexp[jnp.clip(nvalid - 1, 0, _NB - 1)]
    bexp = jnp.where(bid < nvalid, bexp, last).astype(jnp.int32)
    bvalid = (bid < nvalid).astype(jnp.int32)
    bclamp = jnp.minimum(bid, jnp.maximum(nvalid - 1, 0)).astype(jnp.int32)
    return g, p.astype(jnp.int32), bexp, bvalid, bclamp, nvalid


def _meta16(total_chunks):
    return jnp.zeros((16,), jnp.float32).at[0].set(
        jnp.asarray(total_chunks).astype(jnp.float32))


def _i2h_lstm(x, W, b, h2h, c):
    T, D = x.shape
    P = W.shape[1]
    H = c.shape[1]
    BM = 256

    def kern(x_ref, w_ref, b_ref, h2h_ref, c_ref, o_ref):
        gates = (
            jnp.dot(x_ref[...], w_ref[...], preferred_element_type=jnp.float32)
            + b_ref[...]
            + h2h_ref[...]
        )
        i_g = gates[:, 0:H]
        f_g = gates[:, H:2 * H]
        g_g = gates[:, 2 * H:3 * H]
        o_g = gates[:, 3 * H:4 * H]
        c_new = jax.nn.sigmoid(f_g) * c_ref[...] + jax.nn.sigmoid(i_g) * jnp.tanh(g_g)
        h_new = jax.nn.sigmoid(o_g) * jnp.tanh(c_new)
        o_ref[:, 0:H] = h_new
        o_ref[:, H:2 * H] = c_new

    return pl.pallas_call(
        kern,
        grid=(T // BM,),
        in_specs=[
            pl.BlockSpec((BM, D), lambda i: (i, 0)),
            pl.BlockSpec((D, P), lambda i: (0, 0)),
            pl.BlockSpec((1, P), lambda i: (0, 0)),
            pl.BlockSpec((BM, P), lambda i: (i, 0)),
            pl.BlockSpec((BM, H), lambda i: (i, 0)),
        ],
        out_specs=pl.BlockSpec((BM, 2 * H), lambda i: (i, 0)),
        out_shape=jax.ShapeDtypeStruct((T, 2 * H), jnp.float32),
    )(x, W, b.reshape(1, P), h2h, c)


def kernel(x, h, c, W_i2h, b_i2h, Wr0, br0, We0, be0, Wr1, br1, We1, be1):
    T, H = h.shape
    E = We0.shape[0]

    a0 = _router_argmax(h, Wr0, br0, T)[:, 0]
    tok = jnp.arange(T, dtype=jnp.int32)
    g0, p0, emap0, bval0, bcl0, nval0 = _dispatch(
        a0, jnp.ones((T,), jnp.bool_), tok, E)

    hg = _sc_gather(h, g0, _meta16(nval0 * (_BM // 64)), _NP, CH=64)
    big, a1g = _grouped_matmul0(hg, We0, be0, Wr1, br1, emap0, bval0, bcl0)

    a1 = a1g[p0, 0]
    maskv = a0 != 0

    g1, p1, emap1, bval1, bcl1, nval1 = _dispatch(a1, maskv, p0, E)
    ig = _sc_gather(big, g1, _meta16(nval1 * (_BM // 16)), _NP, CH=16)
    big = _grouped_matmul1(ig, We1, be1, emap1, bval1, bcl1, alias_in=big)

    src = jnp.where(maskv, _NP + p1, p0)
    h2h = _sc_gather(big, src, _meta16(jnp.asarray(T // 16)), T, CH=16)
    return _i2h_lstm(x, W_i2h, b_i2h, h2h, c)
